# baseline (speedup 1.0000x reference)
import functools

import numpy as np
import jax
import jax.numpy as jnp
from jax import lax
from jax.experimental import pallas as pl
from jax.experimental.pallas import tpu as pltpu
from jax.experimental.pallas import tpu_sc as plsc

_TD = 16
_NB = 22
_MAXD = 40.0
_N = 2048
_NC, _NS, _L = 2, 16, 16
_NW = _NC * _NS
_ROWS_PER = _N // _NW
_CHUNKS = _N // _L
_BW = np.float32(_MAXD / (_NB - 1))
_INV_BW = float(1.0 / np.float64(_BW))

_MESH = plsc.VectorSubcoreMesh(
    core_axis_name="c", subcore_axis_name="s", num_cores=_NC, num_subcores=_NS
)


@functools.partial(
    pl.kernel,
    out_type=jax.ShapeDtypeStruct((_NB * _TD,), jnp.float32),
    mesh=_MESH,
    compiler_params=pltpu.CompilerParams(needs_layout_passes=False, use_tc_tiling_on_sc=False),
    scratch_types=[
        pltpu.VMEM((_TD * _NB,), jnp.float32),
        pltpu.VMEM((_TD,), jnp.float32),
        pltpu.VMEM((_NB * _TD,), jnp.float32),
    ],
)
def _build_table(w_hbm, b_hbm, t_hbm, w_v, b_v, t_v):
    wid = lax.axis_index("s") * _NC + lax.axis_index("c")

    @pl.when(wid == 0)
    def _():
        pltpu.sync_copy(w_hbm, w_v)
        pltpu.sync_copy(b_hbm, b_v)
        bvec = b_v[...]
        rows = lax.iota(jnp.int32, _L) * _NB
        for t in range(_NB):
            col = plsc.load_gather(w_v, [rows + t])
            t_v[pl.ds(t * _TD, _TD)] = col + bvec
        pltpu.sync_copy(t_v, t_hbm)


@functools.partial(
    pl.kernel,
    out_type=jax.ShapeDtypeStruct((_N, _N, _TD), jnp.float32),
    mesh=_MESH,
    compiler_params=pltpu.CompilerParams(needs_layout_passes=False, use_tc_tiling_on_sc=False),
    scratch_types=[
        pltpu.VMEM((3 * _N,), jnp.float32),
        pltpu.VMEM((_NB * _TD,), jnp.float32),
        pltpu.VMEM((_N, _TD), jnp.float32),
        pltpu.VMEM((_N, _TD), jnp.float32),
        pltpu.SemaphoreType.DMA,
        pltpu.SemaphoreType.DMA,
    ],
)
def _encode(ct_hbm, t_hbm, out_hbm, cxyz_v, tf_v, rows0_v, rows1_v, sem0, sem1):
    wid = lax.axis_index("s") * _NC + lax.axis_index("c")
    pltpu.sync_copy(ct_hbm, cxyz_v)
    pltpu.sync_copy(t_hbm, tf_v)
    base = wid * _ROWS_PER
    iota16 = lax.iota(jnp.int32, _L)

    def do_row(i, rows_v):
        icol = jnp.full((_L,), i, jnp.int32)
        xi = plsc.load_gather(cxyz_v, [icol])
        yi = plsc.load_gather(cxyz_v, [icol + _N])
        zi = plsc.load_gather(cxyz_v, [icol + 2 * _N])

        def chunk_body(c2):
            cbase = c2 * _L
            xj = cxyz_v[pl.ds(cbase, _L)]
            yj = cxyz_v[pl.ds(_N + cbase, _L)]
            zj = cxyz_v[pl.ds(2 * _N + cbase, _L)]
            dx = xj - xi
            dy = yj - yi
            dz = zj - zi
            u = dx * dx + dy * dy + dz * dz + 1e-8
            r = plsc.bitcast(0x5F3759DF - (plsc.bitcast(u, jnp.int32) >> 1),
                             jnp.float32)
            r = r * (1.5 - 0.5 * u * r * r)
            r = r * (1.5 - 0.5 * u * r * r)
            r = r * (1.5 - 0.5 * u * r * r)
            q = u * r * _INV_BW
            ti = q.astype(jnp.int32)
            idx = ti + jnp.where(ti.astype(jnp.float32) < q, 1, 0).astype(jnp.int32)
            gbase = jnp.minimum(idx, _NB - 1) * _TD
            qvec = cbase + iota16
            for k in range(_TD):
                vals = plsc.load_gather(tf_v, [gbase + k])
                plsc.store_scatter(rows_v, [qvec, jnp.full((_L,), k, jnp.int32)], vals)

        plsc.parallel_loop(0, _CHUNKS, 1, unroll=4)(chunk_body)

    def pair_body(r2, carry):
        i0 = base + r2 * 2

        @pl.when(r2 >= 1)
        def _():
            pltpu.make_async_copy(out_hbm.at[0], rows0_v, sem0).wait()
            pltpu.make_async_copy(out_hbm.at[0], rows1_v, sem1).wait()

        do_row(i0, rows0_v)
        pltpu.async_copy(rows0_v, out_hbm.at[i0], sem0)
        do_row(i0 + 1, rows1_v)
        pltpu.async_copy(rows1_v, out_hbm.at[i0 + 1], sem1)
        return carry

    lax.fori_loop(0, _ROWS_PER // 2, pair_body, 0)
    pltpu.make_async_copy(out_hbm.at[0], rows0_v, sem0).wait()
    pltpu.make_async_copy(out_hbm.at[0], rows1_v, sem1).wait()


def kernel(coords, W, b):
    ct = coords.T.reshape(-1)
    table = _build_table(W.reshape(-1), b)
    return _encode(ct, table)
